# baseline (speedup 1.0000x reference)
import functools

import jax
import jax.numpy as jnp
from jax import lax
from jax.experimental import pallas as pl
from jax.experimental.pallas import tpu as pltpu
from jax.experimental.pallas import tpu_sc as plsc

VOCAB = 100000
EMBED_DIMENSION = 300
BATCH = 1024
QUAD = 4
DQ = EMBED_DIMENSION * QUAD
VQ = VOCAB // QUAD
BLOCK_N = 2048


def _gather_quads_sc(table_quad, qidx):
    info = plsc.get_sparse_core_info()
    nw = info.num_cores * info.num_subcores
    b_per_w = BATCH // nw
    mesh = plsc.VectorSubcoreMesh(core_axis_name="c", subcore_axis_name="s")

    @functools.partial(
        pl.kernel,
        mesh=mesh,
        out_type=jax.ShapeDtypeStruct((BATCH, DQ), jnp.float32),
        scratch_types=[
            pltpu.VMEM((b_per_w,), jnp.int32),
            pltpu.VMEM((b_per_w, DQ), jnp.float32),
            pltpu.SemaphoreType.DMA,
        ],
        compiler_params=pltpu.CompilerParams(use_tc_tiling_on_sc=False),
    )
    def gather_kernel(table_hbm, qidx_hbm, out_hbm, idx_v, rows_v, sem):
        wid = lax.axis_index("s") * info.num_cores + lax.axis_index("c")
        base = wid * b_per_w
        pltpu.sync_copy(qidx_hbm.at[pl.ds(base, b_per_w)], idx_v)
        pltpu.async_copy(table_hbm.at[idx_v], rows_v, sem).wait()
        pltpu.sync_copy(rows_v, out_hbm.at[pl.ds(base, b_per_w)])

    return gather_kernel(table_quad, qidx)


def _project_kernel(xq_ref, q_ref, w_ref, b_ref, o_ref):
    xq = xq_ref[...]
    q = q_ref[...]
    x = jnp.where(
        q == 0,
        xq[:, 0:300],
        jnp.where(q == 1, xq[:, 300:600], jnp.where(q == 2, xq[:, 600:900], xq[:, 900:1200])),
    )
    ss = jnp.sum(x * x, axis=1, keepdims=True)
    norm = jnp.sqrt(ss)
    scale = jnp.minimum(1.0, 1.0 / jnp.maximum(norm, 1e-7))
    xs = x * scale
    acc = lax.dot_general(
        xs,
        w_ref[...],
        (((1,), (1,)), ((), ())),
        preferred_element_type=jnp.float32,
    )
    o_ref[...] = acc + b_ref[...]


def _project(xq, q, W, b):
    grid = (pl.cdiv(VOCAB, BLOCK_N),)
    return pl.pallas_call(
        _project_kernel,
        grid=grid,
        in_specs=[
            pl.BlockSpec((BATCH, DQ), lambda j: (0, 0)),
            pl.BlockSpec((BATCH, 1), lambda j: (0, 0)),
            pl.BlockSpec((BLOCK_N, EMBED_DIMENSION), lambda j: (j, 0)),
            pl.BlockSpec((1, BLOCK_N), lambda j: (0, j)),
        ],
        out_specs=pl.BlockSpec((BATCH, BLOCK_N), lambda j: (0, j)),
        out_shape=jax.ShapeDtypeStruct((BATCH, VOCAB), jnp.float32),
    )(xq, q, W, b.reshape(1, VOCAB))


@jax.jit
def kernel(inputs_, emb_table, W, b):
    idx = inputs_.astype(jnp.int32)
    table_quad = emb_table.reshape(VQ, DQ)
    xq = _gather_quads_sc(table_quad, idx >> 2)
    q = (idx & 3).reshape(BATCH, 1)
    return _project(xq, q, W, b)


# speedup vs baseline: 6.0460x; 6.0460x over previous
import functools

import jax
import jax.numpy as jnp
from jax import lax
from jax.experimental import pallas as pl
from jax.experimental.pallas import tpu as pltpu
from jax.experimental.pallas import tpu_sc as plsc

VOCAB = 100000
EMBED_DIMENSION = 300
BATCH = 1024
BLOCK_N = 2048


def _gather_cols_sc(table_t, idx):
    info = plsc.get_sparse_core_info()
    nw = info.num_cores * info.num_subcores
    rows_per_w = (EMBED_DIMENSION + nw - 1) // nw
    mesh = plsc.VectorSubcoreMesh(core_axis_name="c", subcore_axis_name="s")

    @functools.partial(
        pl.kernel,
        mesh=mesh,
        out_type=jax.ShapeDtypeStruct((EMBED_DIMENSION, BATCH), jnp.float32),
        scratch_types=[
            pltpu.VMEM((BATCH,), jnp.int32),
            pltpu.VMEM((VOCAB,), jnp.float32),
            pltpu.VMEM((BATCH,), jnp.float32),
        ],
        compiler_params=pltpu.CompilerParams(needs_layout_passes=False),
    )
    def extract_kernel(et_hbm, idx_hbm, xt_hbm, idx_v, row_v, out_v):
        wid = lax.axis_index("s") * info.num_cores + lax.axis_index("c")
        pltpu.sync_copy(idx_hbm, idx_v)
        for k in range(rows_per_w):
            d = wid + nw * k

            @pl.when(d < EMBED_DIMENSION)
            def _():
                pltpu.sync_copy(et_hbm.at[d], row_v)

                def body(c, carry):
                    chunk = idx_v[pl.ds(c * 16, 16)]
                    vals = plsc.load_gather(row_v, [chunk])
                    out_v[pl.ds(c * 16, 16)] = vals
                    return carry

                lax.fori_loop(0, BATCH // 16, body, 0)
                pltpu.sync_copy(out_v, xt_hbm.at[d])

    return extract_kernel(table_t, idx)


def _project_kernel(xt_ref, wt_ref, b_ref, o_ref):
    xt = xt_ref[...]
    ss = jnp.sum(xt * xt, axis=0, keepdims=True)
    norm = jnp.sqrt(ss)
    scale = jnp.minimum(1.0, 1.0 / jnp.maximum(norm, 1e-7))
    xs = xt * scale
    acc = lax.dot_general(
        wt_ref[...],
        xs,
        (((0,), (0,)), ((), ())),
        preferred_element_type=jnp.float32,
    )
    o_ref[...] = acc + b_ref[...]


def _project(xt, w_t, b):
    grid = (pl.cdiv(VOCAB, BLOCK_N),)
    return pl.pallas_call(
        _project_kernel,
        grid=grid,
        in_specs=[
            pl.BlockSpec((EMBED_DIMENSION, BATCH), lambda j: (0, 0)),
            pl.BlockSpec((EMBED_DIMENSION, BLOCK_N), lambda j: (0, j)),
            pl.BlockSpec((BLOCK_N, 1), lambda j: (j, 0)),
        ],
        out_specs=pl.BlockSpec((BLOCK_N, BATCH), lambda j: (j, 0)),
        out_shape=jax.ShapeDtypeStruct((VOCAB, BATCH), jnp.float32),
    )(xt, w_t, b.reshape(VOCAB, 1))


@jax.jit
def kernel(inputs_, emb_table, W, b):
    idx = inputs_.astype(jnp.int32)
    xt = _gather_cols_sc(emb_table.T, idx)
    out_t = _project(xt, W.T, b)
    return out_t.T


# speedup vs baseline: 6.0488x; 1.0005x over previous
import functools

import jax
import jax.numpy as jnp
from jax import lax
from jax.experimental import pallas as pl
from jax.experimental.pallas import tpu as pltpu
from jax.experimental.pallas import tpu_sc as plsc

VOCAB = 100000
EMBED_DIMENSION = 300
BATCH = 1024
BLOCK_N = 2048


def _gather_cols_sc(table_t, idx):
    info = plsc.get_sparse_core_info()
    nw = info.num_cores * info.num_subcores
    rows_per_w = (EMBED_DIMENSION + nw - 1) // nw
    mesh = plsc.VectorSubcoreMesh(core_axis_name="c", subcore_axis_name="s")

    @functools.partial(
        pl.kernel,
        mesh=mesh,
        out_type=jax.ShapeDtypeStruct((EMBED_DIMENSION, BATCH), jnp.float32),
        scratch_types=[
            pltpu.VMEM((BATCH,), jnp.int32),
            pltpu.VMEM((VOCAB,), jnp.float32),
            pltpu.VMEM((BATCH,), jnp.float32),
        ],
        compiler_params=pltpu.CompilerParams(needs_layout_passes=False),
    )
    def extract_kernel(et_hbm, idx_hbm, xt_hbm, idx_v, row_v, out_v):
        wid = lax.axis_index("s") * info.num_cores + lax.axis_index("c")
        pltpu.sync_copy(idx_hbm, idx_v)
        for k in range(rows_per_w):
            d = wid + nw * k

            @pl.when(d < EMBED_DIMENSION)
            def _():
                pltpu.sync_copy(et_hbm.at[d], row_v)

                def body(c, carry):
                    chunk = idx_v[pl.ds(c * 16, 16)]
                    vals = plsc.load_gather(row_v, [chunk])
                    out_v[pl.ds(c * 16, 16)] = vals
                    return carry

                lax.fori_loop(0, BATCH // 16, body, 0)
                pltpu.sync_copy(out_v, xt_hbm.at[d])

    return extract_kernel(table_t, idx)


def _project_kernel(xt_ref, wt_ref, b_ref, o_ref):
    xt = xt_ref[...]
    ss = jnp.sum(xt * xt, axis=0, keepdims=True)
    norm = jnp.sqrt(ss)
    scale = jnp.minimum(1.0, 1.0 / jnp.maximum(norm, 1e-7))
    xs = (xt * scale).astype(jnp.bfloat16)
    acc = lax.dot_general(
        wt_ref[...].astype(jnp.bfloat16),
        xs,
        (((0,), (0,)), ((), ())),
        preferred_element_type=jnp.float32,
    )
    o_ref[...] = acc + b_ref[...]


def _project(xt, w_t, b):
    grid = (pl.cdiv(VOCAB, BLOCK_N),)
    return pl.pallas_call(
        _project_kernel,
        grid=grid,
        in_specs=[
            pl.BlockSpec((EMBED_DIMENSION, BATCH), lambda j: (0, 0)),
            pl.BlockSpec((EMBED_DIMENSION, BLOCK_N), lambda j: (0, j)),
            pl.BlockSpec((BLOCK_N, 1), lambda j: (j, 0)),
        ],
        out_specs=pl.BlockSpec((BLOCK_N, BATCH), lambda j: (j, 0)),
        out_shape=jax.ShapeDtypeStruct((VOCAB, BATCH), jnp.float32),
    )(xt, w_t, b.reshape(VOCAB, 1))


@jax.jit
def kernel(inputs_, emb_table, W, b):
    idx = inputs_.astype(jnp.int32)
    xt = _gather_cols_sc(emb_table.T, idx)
    out_t = _project(xt, W.T, b)
    return out_t.T


# speedup vs baseline: 6.1802x; 1.0217x over previous
import functools

import jax
import jax.numpy as jnp
from jax import lax
from jax.experimental import pallas as pl
from jax.experimental.pallas import tpu as pltpu
from jax.experimental.pallas import tpu_sc as plsc

VOCAB = 100000
EMBED_DIMENSION = 300
BATCH = 1024
BLOCK_N = 4096


def _gather_cols_sc(table_t, idx):
    info = plsc.get_sparse_core_info()
    nw = info.num_cores * info.num_subcores
    rows_per_w = (EMBED_DIMENSION + nw - 1) // nw
    mesh = plsc.VectorSubcoreMesh(core_axis_name="c", subcore_axis_name="s")

    @functools.partial(
        pl.kernel,
        mesh=mesh,
        out_type=jax.ShapeDtypeStruct((EMBED_DIMENSION, BATCH), jnp.float32),
        scratch_types=[
            pltpu.VMEM((BATCH,), jnp.int32),
            pltpu.VMEM((VOCAB,), jnp.float32),
            pltpu.VMEM((BATCH,), jnp.float32),
        ],
        compiler_params=pltpu.CompilerParams(needs_layout_passes=False),
    )
    def extract_kernel(et_hbm, idx_hbm, xt_hbm, idx_v, row_v, out_v):
        wid = lax.axis_index("s") * info.num_cores + lax.axis_index("c")
        pltpu.sync_copy(idx_hbm, idx_v)
        for k in range(rows_per_w):
            d = wid + nw * k

            @pl.when(d < EMBED_DIMENSION)
            def _():
                pltpu.sync_copy(et_hbm.at[d], row_v)

                def body(c, carry):
                    chunk = idx_v[pl.ds(c * 16, 16)]
                    vals = plsc.load_gather(row_v, [chunk])
                    out_v[pl.ds(c * 16, 16)] = vals
                    return carry

                lax.fori_loop(0, BATCH // 16, body, 0)
                pltpu.sync_copy(out_v, xt_hbm.at[d])

    return extract_kernel(table_t, idx)


def _project_kernel(xt_ref, wt_ref, b_ref, o_ref):
    xt = xt_ref[...]
    ss = jnp.sum(xt * xt, axis=0, keepdims=True)
    norm = jnp.sqrt(ss)
    scale = jnp.minimum(1.0, 1.0 / jnp.maximum(norm, 1e-7))
    xs = (xt * scale).astype(jnp.bfloat16)
    acc = lax.dot_general(
        wt_ref[...].astype(jnp.bfloat16),
        xs,
        (((0,), (0,)), ((), ())),
        preferred_element_type=jnp.float32,
    )
    o_ref[...] = acc + b_ref[...]


def _project(xt, w_t, b):
    grid = (pl.cdiv(VOCAB, BLOCK_N),)
    return pl.pallas_call(
        _project_kernel,
        grid=grid,
        in_specs=[
            pl.BlockSpec((EMBED_DIMENSION, BATCH), lambda j: (0, 0)),
            pl.BlockSpec((EMBED_DIMENSION, BLOCK_N), lambda j: (0, j)),
            pl.BlockSpec((BLOCK_N, 1), lambda j: (j, 0)),
        ],
        out_specs=pl.BlockSpec((BLOCK_N, BATCH), lambda j: (j, 0)),
        out_shape=jax.ShapeDtypeStruct((VOCAB, BATCH), jnp.float32),
    )(xt, w_t, b.reshape(VOCAB, 1))


@jax.jit
def kernel(inputs_, emb_table, W, b):
    idx = inputs_.astype(jnp.int32)
    xt = _gather_cols_sc(emb_table.T, idx)
    out_t = _project(xt, W.T, b)
    return out_t.T
